# baseline (speedup 1.0000x reference)
import functools
import math

import jax
import jax.numpy as jnp
from jax import lax
from jax.experimental import pallas as pl
from jax.experimental.pallas import tpu as pltpu
from jax.experimental.pallas import tpu_sc as plsc

D = 32
SCALE = math.sqrt(D)

_info = plsc.get_sparse_core_info()
NC, NS, L = _info.num_cores, _info.num_subcores, _info.num_lanes
NW = NC * NS

BATCH = 4096
SEQ = 200
BC = BATCH // NW
TS = 4
NSTEP = SEQ // TS
PITCH = BC + 1
NBUF = 3
Y5_SHAPE = (SEQ, D // 8, NW, 8, BC)


VOCAB = 1000000
CV = 800
NCH = VOCAB // CV
KMAX = -(-NCH // NW)
VP = 33


def _tr_tec(inbuf_ref, obuf_ref):
    iota = lax.iota(jnp.int32, L)

    def v_step(g, _):
        idx_v = iota + g * L
        for d in range(D):
            r = inbuf_ref[d, pl.ds(g * L, L)]
            plsc.store_scatter(obuf_ref, [idx_v, jnp.full((L,), d, jnp.int32)], r)
        return 0

    lax.fori_loop(0, CV // L, v_step, 0, unroll=2)


def _tr_body(lutT_hbm, l_hbm, inbufs, obufs, isems, osems):
    wid = lax.axis_index("s") * NC + lax.axis_index("c")

    def chunk(k):
        return jnp.minimum(k * NW + wid, NCH - 1)

    def ipair(k, slot):
        return lutT_hbm.at[:, pl.ds(chunk(k) * CV, CV)], inbufs[slot], isems[slot]

    def opair(k, slot):
        return (
            obufs[slot].at[:, pl.ds(0, D)],
            l_hbm.at[pl.ds(chunk(k) * CV, CV), :],
            osems[slot],
        )

    pltpu.async_copy(*ipair(0, 0))

    def step(j, _):
        for s in range(2):
            k = 2 * j + s
            pltpu.make_async_copy(*ipair(k, s)).wait()

            @pl.when(k < KMAX - 1)
            def _():
                pltpu.async_copy(*ipair(k + 1, 1 - s))

            @pl.when(k >= 2)
            def _():
                pltpu.make_async_copy(*opair(k - 2, s)).wait()

            _tr_tec(inbufs[s], obufs[s])
            pltpu.async_copy(*opair(k, s))

        return 0

    lax.fori_loop(0, KMAX // 2, step, 0)
    for k in range(KMAX - 2, KMAX):
        pltpu.make_async_copy(*opair(k, k % 2)).wait()


@jax.jit
def _transpose(lutT):
    mesh = plsc.VectorSubcoreMesh(core_axis_name="c", subcore_axis_name="s")
    f = functools.partial(
        pl.kernel,
        mesh=mesh,
        out_type=jax.ShapeDtypeStruct((VOCAB, D), jnp.float32),
        scratch_types=[
            [pltpu.VMEM((D, CV), jnp.float32) for _ in range(2)],
            [pltpu.VMEM((CV, VP), jnp.float32) for _ in range(2)],
            [pltpu.SemaphoreType.DMA for _ in range(2)],
            [pltpu.SemaphoreType.DMA for _ in range(2)],
        ],
        compiler_params=pltpu.CompilerParams(
            use_tc_tiling_on_sc=False, needs_layout_passes=False
        ),
    )(_tr_body)
    return f(lutT)


def _tec_transform(rows_ref, ybuf_ref):
    d = lax.iota(jnp.int32, L)
    i_dgrp0 = d >> 3
    i_dd = d & 7
    i_dgrp1 = i_dgrp0 + 2
    zero = jnp.full((L,), 0, jnp.int32)

    for tl in range(TS):
        i_tl = jnp.full((L,), tl, jnp.int32)

        def row_step(bb, _):
            i_bb = zero + bb
            r0 = rows_ref[tl * BC + bb, pl.ds(0, L)]
            plsc.store_scatter(ybuf_ref, [i_tl, i_dgrp0, i_dd, i_bb], r0 * SCALE)
            r1 = rows_ref[tl * BC + bb, pl.ds(L, L)]
            plsc.store_scatter(ybuf_ref, [i_tl, i_dgrp1, i_dd, i_bb], r1 * SCALE)
            return 0

        lax.fori_loop(0, BC, row_step, 0, unroll=8)


def _emb_body(xg_hbm, lut_hbm, y5_hbm, xblk, rows, ybufs, gsems, wsems):
    wid = lax.axis_index("s") * NC + lax.axis_index("c")

    pltpu.sync_copy(xg_hbm.at[wid], xblk)

    def gather_start(k, slot):
        pltpu.async_copy(lut_hbm.at[xblk.at[pl.ds(k * TS * BC, TS * BC)]],
                         rows[slot], gsems[slot])

    def gather_wait(k, slot):
        pltpu.make_async_copy(lut_hbm.at[xblk.at[pl.ds(k * TS * BC, TS * BC)]],
                              rows[slot], gsems[slot]).wait()

    def wpair(k, slot):
        src = ybufs[slot].at[:, :, :, pl.ds(0, BC)]
        dst = y5_hbm.at[pl.ds(k * TS, TS), :, wid]
        return src, dst

    def write_start(k, slot):
        src, dst = wpair(k, slot)
        pltpu.async_copy(src, dst, wsems[slot])

    def write_wait(k, slot):
        src, dst = wpair(k, slot)
        pltpu.make_async_copy(src, dst, wsems[slot]).wait()

    for s in range(NBUF):
        gather_start(s, s)

    def step(j, _):
        for s in range(NBUF):
            k = NBUF * j + s
            gather_wait(k, s)

            @pl.when(k >= NBUF)
            def _():
                write_wait(k - NBUF, s)

            _tec_transform(rows[s], ybufs[s])
            write_start(k, s)

            @pl.when(k < NSTEP - NBUF)
            def _():
                gather_start(k + NBUF, s)

        return 0

    lax.fori_loop(0, NSTEP // NBUF, step, 0)
    for k in range(NSTEP // NBUF * NBUF, NSTEP):
        s = k % NBUF
        gather_wait(k, s)
        write_wait(k - NBUF, s)
        _tec_transform(rows[s], ybufs[s])
        write_start(k, s)

    for k in range(NSTEP - NBUF, NSTEP):
        write_wait(k, k % NBUF)


@jax.jit
def _emb(xg, lut):
    mesh = plsc.VectorSubcoreMesh(core_axis_name="c", subcore_axis_name="s")
    f = functools.partial(
        pl.kernel,
        mesh=mesh,
        out_type=jax.ShapeDtypeStruct(Y5_SHAPE, jnp.float32),
        scratch_types=[
            pltpu.VMEM((SEQ * BC,), jnp.int32),
            [pltpu.VMEM((TS * BC, D), jnp.float32) for _ in range(NBUF)],
            [pltpu.VMEM((TS, D // 8, 8, PITCH), jnp.float32) for _ in range(NBUF)],
            [pltpu.SemaphoreType.DMA for _ in range(NBUF)],
            [pltpu.SemaphoreType.DMA for _ in range(NBUF)],
        ],
        compiler_params=pltpu.CompilerParams(
            use_tc_tiling_on_sc=False, needs_layout_passes=False
        ),
    )(_emb_body)
    return f(xg, lut)


def kernel(x, lut):
    xg = (
        x.T.astype(jnp.int32)
        .reshape(SEQ, NW, BC)
        .transpose(1, 0, 2)
        .reshape(NW, SEQ * BC)
    )
    l_lin = _transpose(lut.T)
    y5 = _emb(xg, l_lin)
    return jnp.transpose(y5, (2, 4, 0, 1, 3)).reshape(BATCH, SEQ, D)


# speedup vs baseline: 3.4649x; 3.4649x over previous
import functools
import math

import jax
import jax.numpy as jnp
from jax import lax
from jax.experimental import pallas as pl
from jax.experimental.pallas import tpu as pltpu
from jax.experimental.pallas import tpu_sc as plsc

D = 32
SCALE = math.sqrt(D)

_info = plsc.get_sparse_core_info()
NC, NS, L = _info.num_cores, _info.num_subcores, _info.num_lanes
NW = NC * NS

BATCH = 4096
SEQ = 200
BC = BATCH // NW
TS = 4
NSTEP = SEQ // TS
PITCH = BC + 1
NBUF = 3
Y5_SHAPE = (SEQ, D // 8, NW, 8, BC)


VOCAB = 1000000
QS = 250112
BLKC = 256
GRID = QS // BLKC
VCAP = 4 * QS


CLAST = (VOCAB - 3 * QS) // BLKC


def _tr_tc_body(a0, a1, a2, a3, tail, o):
    c = pl.program_id(0)
    o[...] = jnp.concatenate(
        [a0[...].T, a1[...].T, a2[...].T, a3[...].T], axis=1
    )

    @pl.when(c == CLAST)
    def _():
        o[pl.ds(0, VOCAB - 3 * QS - CLAST * BLKC), pl.ds(3 * D, D)] = tail[...]


@jax.jit
def _transpose(lutT, tail64):
    specs = [
        pl.BlockSpec((D, BLKC), (lambda c, q=q: (0, q * GRID + c)))
        for q in range(3)
    ]
    specs.append(
        pl.BlockSpec(
            (D, BLKC),
            lambda c: (0, jnp.where(c < CLAST, 3 * GRID + c, 0)),
        )
    )
    specs.append(pl.BlockSpec((VOCAB - 3 * QS - CLAST * BLKC, D), lambda c: (0, 0)))
    return pl.pallas_call(
        _tr_tc_body,
        grid=(GRID,),
        in_specs=specs,
        out_specs=pl.BlockSpec((BLKC, 4 * D), lambda c: (c, 0)),
        out_shape=jax.ShapeDtypeStruct((QS, 4 * D), jnp.float32),
    )(lutT, lutT, lutT, lutT, tail64)


def _tec_transform(rows_ref, ybuf_ref):
    d = lax.iota(jnp.int32, L)
    i_dgrp0 = d >> 3
    i_dd = d & 7
    i_dgrp1 = i_dgrp0 + 2
    zero = jnp.full((L,), 0, jnp.int32)

    for tl in range(TS):
        i_tl = jnp.full((L,), tl, jnp.int32)

        def row_step(bb, _):
            i_bb = zero + bb
            r0 = rows_ref[tl * BC + bb, pl.ds(0, L)]
            plsc.store_scatter(ybuf_ref, [i_tl, i_dgrp0, i_dd, i_bb], r0 * SCALE)
            r1 = rows_ref[tl * BC + bb, pl.ds(L, L)]
            plsc.store_scatter(ybuf_ref, [i_tl, i_dgrp1, i_dd, i_bb], r1 * SCALE)
            return 0

        lax.fori_loop(0, BC, row_step, 0, unroll=8)


def _emb_body(xg_hbm, lut_hbm, y5_hbm, xblk, rows, ybufs, gsems, wsems):
    wid = lax.axis_index("s") * NC + lax.axis_index("c")

    pltpu.sync_copy(xg_hbm.at[wid], xblk)

    def gather_start(k, slot):
        pltpu.async_copy(lut_hbm.at[xblk.at[pl.ds(k * TS * BC, TS * BC)]],
                         rows[slot], gsems[slot])

    def gather_wait(k, slot):
        pltpu.make_async_copy(lut_hbm.at[xblk.at[pl.ds(k * TS * BC, TS * BC)]],
                              rows[slot], gsems[slot]).wait()

    def wpair(k, slot):
        src = ybufs[slot].at[:, :, :, pl.ds(0, BC)]
        dst = y5_hbm.at[pl.ds(k * TS, TS), :, wid]
        return src, dst

    def write_start(k, slot):
        src, dst = wpair(k, slot)
        pltpu.async_copy(src, dst, wsems[slot])

    def write_wait(k, slot):
        src, dst = wpair(k, slot)
        pltpu.make_async_copy(src, dst, wsems[slot]).wait()

    for s in range(NBUF):
        gather_start(s, s)

    def step(j, _):
        for s in range(NBUF):
            k = NBUF * j + s
            gather_wait(k, s)

            @pl.when(k >= NBUF)
            def _():
                write_wait(k - NBUF, s)

            _tec_transform(rows[s], ybufs[s])
            write_start(k, s)

            @pl.when(k < NSTEP - NBUF)
            def _():
                gather_start(k + NBUF, s)

        return 0

    lax.fori_loop(0, NSTEP // NBUF, step, 0)
    for k in range(NSTEP // NBUF * NBUF, NSTEP):
        s = k % NBUF
        gather_wait(k, s)
        write_wait(k - NBUF, s)
        _tec_transform(rows[s], ybufs[s])
        write_start(k, s)

    for k in range(NSTEP - NBUF, NSTEP):
        write_wait(k, k % NBUF)


@jax.jit
def _emb(xg, lut):
    mesh = plsc.VectorSubcoreMesh(core_axis_name="c", subcore_axis_name="s")
    f = functools.partial(
        pl.kernel,
        mesh=mesh,
        out_type=jax.ShapeDtypeStruct(Y5_SHAPE, jnp.float32),
        scratch_types=[
            pltpu.VMEM((SEQ * BC,), jnp.int32),
            [pltpu.VMEM((TS * BC, D), jnp.float32) for _ in range(NBUF)],
            [pltpu.VMEM((TS, D // 8, 8, PITCH), jnp.float32) for _ in range(NBUF)],
            [pltpu.SemaphoreType.DMA for _ in range(NBUF)],
            [pltpu.SemaphoreType.DMA for _ in range(NBUF)],
        ],
        compiler_params=pltpu.CompilerParams(
            use_tc_tiling_on_sc=False, needs_layout_passes=False
        ),
    )(_emb_body)
    return f(xg, lut)


def kernel(x, lut):
    xi = x.astype(jnp.int32)
    xr = 4 * (xi % QS) + xi // QS
    xg = xr.T.reshape(SEQ, NW, BC).transpose(1, 0, 2).reshape(NW, SEQ * BC)
    tail64 = lax.slice(lut, (3 * QS + CLAST * BLKC, 0), (VOCAB, D))
    l_lin = _transpose(lut.T, tail64).reshape(VCAP, D)
    y5 = _emb(xg, l_lin)
    return jnp.transpose(y5, (2, 4, 0, 1, 3)).reshape(BATCH, SEQ, D)


# speedup vs baseline: 5.9993x; 1.7314x over previous
import functools
import math

import jax
import jax.numpy as jnp
from jax import lax
from jax.experimental import pallas as pl
from jax.experimental.pallas import tpu as pltpu
from jax.experimental.pallas import tpu_sc as plsc

D = 32
SCALE = math.sqrt(D)

_info = plsc.get_sparse_core_info()
NC, NS, L = _info.num_cores, _info.num_subcores, _info.num_lanes
NW = NC * NS

BATCH = 4096
SEQ = 200
BC = BATCH // NW
TS = 4
NSTEP = SEQ // TS
PITCH = BC + 1
NBUF = 3
Y5_SHAPE = (SEQ, D // 8, NW, 8, BC)


VOCAB = 1000000
BLKC = 2048
QS = BLKC * 123
GRID = QS // BLKC
VCAP = 4 * QS


CLAST = (VOCAB - 3 * QS) // BLKC


def _tr_tc_body(a0, a1, a2, a3, tail, o):
    c = pl.program_id(0)
    o[...] = jnp.concatenate(
        [a0[...].T, a1[...].T, a2[...].T, a3[...].T], axis=1
    )

    @pl.when(c == CLAST)
    def _():
        o[pl.ds(0, VOCAB - 3 * QS - CLAST * BLKC), pl.ds(3 * D, D)] = tail[...]


@jax.jit
def _transpose(lutT, tail64):
    specs = [
        pl.BlockSpec((D, BLKC), (lambda c, q=q: (0, q * GRID + c)))
        for q in range(3)
    ]
    specs.append(
        pl.BlockSpec(
            (D, BLKC),
            lambda c: (0, jnp.where(c < CLAST, 3 * GRID + c, 0)),
        )
    )
    specs.append(pl.BlockSpec((VOCAB - 3 * QS - CLAST * BLKC, D), lambda c: (0, 0)))
    return pl.pallas_call(
        _tr_tc_body,
        grid=(GRID,),
        in_specs=specs,
        out_specs=pl.BlockSpec((BLKC, 4 * D), lambda c: (c, 0)),
        out_shape=jax.ShapeDtypeStruct((QS, 4 * D), jnp.float32),
    )(lutT, lutT, lutT, lutT, tail64)


def _tec_transform(rows_ref, ybuf_ref):
    d = lax.iota(jnp.int32, L)
    i_dgrp0 = d >> 3
    i_dd = d & 7
    i_dgrp1 = i_dgrp0 + 2
    zero = jnp.full((L,), 0, jnp.int32)

    for tl in range(TS):
        i_tl = jnp.full((L,), tl, jnp.int32)

        def row_step(bb, _):
            i_bb = zero + bb
            r0 = rows_ref[tl * BC + bb, pl.ds(0, L)]
            plsc.store_scatter(ybuf_ref, [i_tl, i_dgrp0, i_dd, i_bb], r0 * SCALE)
            r1 = rows_ref[tl * BC + bb, pl.ds(L, L)]
            plsc.store_scatter(ybuf_ref, [i_tl, i_dgrp1, i_dd, i_bb], r1 * SCALE)
            return 0

        lax.fori_loop(0, BC, row_step, 0, unroll=8)


def _emb_body(xg_hbm, lut_hbm, y5_hbm, xblk, rows, ybufs, gsems, wsems):
    wid = lax.axis_index("s") * NC + lax.axis_index("c")

    pltpu.sync_copy(xg_hbm.at[wid], xblk)

    def gather_start(k, slot):
        pltpu.async_copy(lut_hbm.at[xblk.at[pl.ds(k * TS * BC, TS * BC)]],
                         rows[slot], gsems[slot])

    def gather_wait(k, slot):
        pltpu.make_async_copy(lut_hbm.at[xblk.at[pl.ds(k * TS * BC, TS * BC)]],
                              rows[slot], gsems[slot]).wait()

    def wpair(k, slot):
        src = ybufs[slot].at[:, :, :, pl.ds(0, BC)]
        dst = y5_hbm.at[pl.ds(k * TS, TS), :, wid]
        return src, dst

    def write_start(k, slot):
        src, dst = wpair(k, slot)
        pltpu.async_copy(src, dst, wsems[slot])

    def write_wait(k, slot):
        src, dst = wpair(k, slot)
        pltpu.make_async_copy(src, dst, wsems[slot]).wait()

    for s in range(NBUF):
        gather_start(s, s)

    def step(j, _):
        for s in range(NBUF):
            k = NBUF * j + s
            gather_wait(k, s)

            @pl.when(k >= NBUF)
            def _():
                write_wait(k - NBUF, s)

            _tec_transform(rows[s], ybufs[s])
            write_start(k, s)

            @pl.when(k < NSTEP - NBUF)
            def _():
                gather_start(k + NBUF, s)

        return 0

    lax.fori_loop(0, NSTEP // NBUF, step, 0)
    for k in range(NSTEP // NBUF * NBUF, NSTEP):
        s = k % NBUF
        gather_wait(k, s)
        write_wait(k - NBUF, s)
        _tec_transform(rows[s], ybufs[s])
        write_start(k, s)

    for k in range(NSTEP - NBUF, NSTEP):
        write_wait(k, k % NBUF)


@jax.jit
def _emb(xg, lut):
    mesh = plsc.VectorSubcoreMesh(core_axis_name="c", subcore_axis_name="s")
    f = functools.partial(
        pl.kernel,
        mesh=mesh,
        out_type=jax.ShapeDtypeStruct(Y5_SHAPE, jnp.float32),
        scratch_types=[
            pltpu.VMEM((SEQ * BC,), jnp.int32),
            [pltpu.VMEM((TS * BC, D), jnp.float32) for _ in range(NBUF)],
            [pltpu.VMEM((TS, D // 8, 8, PITCH), jnp.float32) for _ in range(NBUF)],
            [pltpu.SemaphoreType.DMA for _ in range(NBUF)],
            [pltpu.SemaphoreType.DMA for _ in range(NBUF)],
        ],
        compiler_params=pltpu.CompilerParams(
            use_tc_tiling_on_sc=False, needs_layout_passes=False
        ),
    )(_emb_body)
    return f(xg, lut)


def kernel(x, lut):
    xi = x.astype(jnp.int32)
    xr = 4 * (xi % QS) + xi // QS
    xg = xr.T.reshape(SEQ, NW, BC).transpose(1, 0, 2).reshape(NW, SEQ * BC)
    tail64 = lax.slice(lut, (3 * QS + CLAST * BLKC, 0), (VOCAB, D))
    l_lin = _transpose(lut.T, tail64).reshape(VCAP, D)
    y5 = _emb(xg, l_lin)
    return jnp.transpose(y5, (2, 4, 0, 1, 3)).reshape(BATCH, SEQ, D)


# speedup vs baseline: 6.0901x; 1.0151x over previous
import functools
import math

import jax
import jax.numpy as jnp
from jax import lax
from jax.experimental import pallas as pl
from jax.experimental.pallas import tpu as pltpu
from jax.experimental.pallas import tpu_sc as plsc

D = 32
SCALE = math.sqrt(D)

_info = plsc.get_sparse_core_info()
NC, NS, L = _info.num_cores, _info.num_subcores, _info.num_lanes
NW = NC * NS

BATCH = 4096
SEQ = 200
BC = BATCH // NW
TS = 4
NSTEP = SEQ // TS
PITCH = BC + 1
NBUF = 3
Y5_SHAPE = (SEQ, D // 8, NW, 8, BC)


VOCAB = 1000000
BLKC = 4096
QS = BLKC * 62
GRID = QS // BLKC
VCAP = 4 * QS


CLAST = (VOCAB - 3 * QS) // BLKC


def _tr_tc_body(a0, a1, a2, a3, tail, o):
    c = pl.program_id(0)
    o[...] = jnp.concatenate(
        [a0[...].T, a1[...].T, a2[...].T, a3[...].T], axis=1
    )

    @pl.when(c == CLAST)
    def _():
        o[pl.ds(0, VOCAB - 3 * QS - CLAST * BLKC), pl.ds(3 * D, D)] = tail[...]


@jax.jit
def _transpose(lutT, tail64):
    specs = [
        pl.BlockSpec((D, BLKC), (lambda c, q=q: (0, q * GRID + c)))
        for q in range(3)
    ]
    specs.append(
        pl.BlockSpec(
            (D, BLKC),
            lambda c: (0, jnp.where(c < CLAST, 3 * GRID + c, 0)),
        )
    )
    specs.append(pl.BlockSpec((VOCAB - 3 * QS - CLAST * BLKC, D), lambda c: (0, 0)))
    return pl.pallas_call(
        _tr_tc_body,
        grid=(GRID,),
        in_specs=specs,
        out_specs=pl.BlockSpec((BLKC, 4 * D), lambda c: (c, 0)),
        out_shape=jax.ShapeDtypeStruct((QS, 4 * D), jnp.float32),
    )(lutT, lutT, lutT, lutT, tail64)


def _tec_transform(rows_ref, ybuf_ref):
    d = lax.iota(jnp.int32, L)
    i_dgrp0 = d >> 3
    i_dd = d & 7
    i_dgrp1 = i_dgrp0 + 2
    zero = jnp.full((L,), 0, jnp.int32)

    for tl in range(TS):
        i_tl = jnp.full((L,), tl, jnp.int32)

        def row_step(bb, _):
            i_bb = zero + bb
            r0 = rows_ref[tl * BC + bb, pl.ds(0, L)]
            plsc.store_scatter(ybuf_ref, [i_tl, i_dgrp0, i_dd, i_bb], r0 * SCALE)
            r1 = rows_ref[tl * BC + bb, pl.ds(L, L)]
            plsc.store_scatter(ybuf_ref, [i_tl, i_dgrp1, i_dd, i_bb], r1 * SCALE)
            return 0

        lax.fori_loop(0, BC, row_step, 0, unroll=8)


def _emb_body(xg_hbm, lut_hbm, y5_hbm, xblk, rows, ybufs, gsems, wsems):
    wid = lax.axis_index("s") * NC + lax.axis_index("c")

    pltpu.sync_copy(xg_hbm.at[wid], xblk)

    def gather_start(k, slot):
        pltpu.async_copy(lut_hbm.at[xblk.at[pl.ds(k * TS * BC, TS * BC)]],
                         rows[slot], gsems[slot])

    def gather_wait(k, slot):
        pltpu.make_async_copy(lut_hbm.at[xblk.at[pl.ds(k * TS * BC, TS * BC)]],
                              rows[slot], gsems[slot]).wait()

    def wpair(k, slot):
        src = ybufs[slot].at[:, :, :, pl.ds(0, BC)]
        dst = y5_hbm.at[pl.ds(k * TS, TS), :, wid]
        return src, dst

    def write_start(k, slot):
        src, dst = wpair(k, slot)
        pltpu.async_copy(src, dst, wsems[slot])

    def write_wait(k, slot):
        src, dst = wpair(k, slot)
        pltpu.make_async_copy(src, dst, wsems[slot]).wait()

    for s in range(NBUF):
        gather_start(s, s)

    def step(j, _):
        for s in range(NBUF):
            k = NBUF * j + s
            gather_wait(k, s)

            @pl.when(k >= NBUF)
            def _():
                write_wait(k - NBUF, s)

            _tec_transform(rows[s], ybufs[s])
            write_start(k, s)

            @pl.when(k < NSTEP - NBUF)
            def _():
                gather_start(k + NBUF, s)

        return 0

    lax.fori_loop(0, NSTEP // NBUF, step, 0)
    for k in range(NSTEP // NBUF * NBUF, NSTEP):
        s = k % NBUF
        gather_wait(k, s)
        write_wait(k - NBUF, s)
        _tec_transform(rows[s], ybufs[s])
        write_start(k, s)

    for k in range(NSTEP - NBUF, NSTEP):
        write_wait(k, k % NBUF)


@jax.jit
def _emb(xg, lut):
    mesh = plsc.VectorSubcoreMesh(core_axis_name="c", subcore_axis_name="s")
    f = functools.partial(
        pl.kernel,
        mesh=mesh,
        out_type=jax.ShapeDtypeStruct(Y5_SHAPE, jnp.float32),
        scratch_types=[
            pltpu.VMEM((SEQ * BC,), jnp.int32),
            [pltpu.VMEM((TS * BC, D), jnp.float32) for _ in range(NBUF)],
            [pltpu.VMEM((TS, D // 8, 8, PITCH), jnp.float32) for _ in range(NBUF)],
            [pltpu.SemaphoreType.DMA for _ in range(NBUF)],
            [pltpu.SemaphoreType.DMA for _ in range(NBUF)],
        ],
        compiler_params=pltpu.CompilerParams(
            use_tc_tiling_on_sc=False, needs_layout_passes=False
        ),
    )(_emb_body)
    return f(xg, lut)


def kernel(x, lut):
    xi = x.astype(jnp.int32)
    xr = 4 * (xi % QS) + xi // QS
    xg = xr.T.reshape(SEQ, NW, BC).transpose(1, 0, 2).reshape(NW, SEQ * BC)
    tail64 = lax.slice(lut, (3 * QS + CLAST * BLKC, 0), (VOCAB, D))
    l_lin = _transpose(lut.T, tail64).reshape(VCAP, D)
    y5 = _emb(xg, l_lin)
    return jnp.transpose(y5, (2, 4, 0, 1, 3)).reshape(BATCH, SEQ, D)
